# baseline (speedup 1.0000x reference)
import functools

import jax
import jax.numpy as jnp
from jax import lax
from jax.experimental import pallas as pl
from jax.experimental.pallas import tpu as pltpu
from jax.experimental.pallas import tpu_sc as plsc

D = 768
LANES = 16
NV = D // LANES
R = 64


def _build_sc_kernel(n_rows):
    info = plsc.get_sparse_core_info()
    nc, ns = info.num_cores, info.num_subcores
    nw = nc * ns
    rows_w = n_rows // nw
    nchunks = rows_w // R
    mesh = plsc.VectorSubcoreMesh(core_axis_name="c", subcore_axis_name="s")

    @functools.partial(
        pl.kernel,
        mesh=mesh,
        out_type=jax.ShapeDtypeStruct((n_rows, D), jnp.float32),
        scratch_types=[
            pltpu.VMEM((R, D), jnp.float32),
            pltpu.VMEM((R, D), jnp.float32),
            pltpu.VMEM((R,), jnp.int32),
            pltpu.SemaphoreType.DMA,
            pltpu.SemaphoreType.DMA,
        ],
    )
    def sc_kernel(emb_hbm, idx_hbm, tbl_hbm, out_hbm, emb_v, tbl_v, idx_v,
                  sem_e, sem_t):
        wid = lax.axis_index("s") * nc + lax.axis_index("c")
        base = wid * rows_w

        def chunk_body(c, carry):
            rbase = base + c * R
            cp_e = pltpu.async_copy(emb_hbm.at[pl.ds(rbase, R)], emb_v, sem_e)
            pltpu.sync_copy(idx_hbm.at[pl.ds(rbase, R)], idx_v)
            cp_t = pltpu.async_copy(tbl_hbm.at[idx_v], tbl_v, sem_t)
            cp_e.wait()
            cp_t.wait()

            def row_body(r, rc):
                for v in range(NV):
                    sl = pl.ds(v * LANES, LANES)
                    emb_v[r, sl] = emb_v[r, sl] + tbl_v[r, sl] * 0.1
                return rc

            lax.fori_loop(0, R, row_body, 0)
            pltpu.sync_copy(emb_v, out_hbm.at[pl.ds(rbase, R)])
            return carry

        lax.fori_loop(0, nchunks, chunk_body, 0)

    return sc_kernel


def kernel(embeddings, section_indices, section_table):
    b, t, d = embeddings.shape
    n = b * t
    emb2d = embeddings.reshape(n, d)
    idx = section_indices.reshape(n).astype(jnp.int32)
    out = _build_sc_kernel(n)(emb2d, idx, section_table)
    return out.reshape(b, t, d)
